# baseline (speedup 1.0000x reference)
import functools

import jax
import jax.numpy as jnp
from jax import lax
from jax.experimental import pallas as pl
from jax.experimental.pallas import tpu as pltpu
from jax.experimental.pallas import tpu_sc as plsc

ROWS = 128
COLS = 100000
BLOCK_C = 2048
NB = (COLS + BLOCK_C - 1) // BLOCK_C

_KS0 = 0
_KS1 = 42
_KS2 = 42 ^ 0x1BD11BDA

_ROT_A = (13, 15, 26, 6)
_ROT_B = (17, 29, 16, 24)


def _rotl(x, d):
    return lax.shift_left(x, jnp.int32(d)) | lax.shift_right_logical(
        x, jnp.int32(32 - d)
    )


def _rounds(x0, x1, rots):
    for d in rots:
        x0 = x0 + x1
        x1 = x0 ^ _rotl(x1, d)
    return x0, x1


def _threefry_bits(flat_idx):
    ks0 = jnp.int32(_KS0)
    ks1 = jnp.int32(_KS1)
    ks2 = jnp.int32(_KS2)
    x0 = jnp.zeros_like(flat_idx) + ks0
    x1 = flat_idx + ks1
    x0, x1 = _rounds(x0, x1, _ROT_A)
    x0 = x0 + ks1
    x1 = x1 + (ks2 + jnp.int32(1))
    x0, x1 = _rounds(x0, x1, _ROT_B)
    x0 = x0 + ks2
    x1 = x1 + (ks0 + jnp.int32(2))
    x0, x1 = _rounds(x0, x1, _ROT_A)
    x0 = x0 + ks0
    x1 = x1 + (ks1 + jnp.int32(3))
    x0, x1 = _rounds(x0, x1, _ROT_B)
    x0 = x0 + ks1
    x1 = x1 + (ks2 + jnp.int32(4))
    x0, x1 = _rounds(x0, x1, _ROT_A)
    x0 = x0 + ks2
    x1 = x1 + (ks0 + jnp.int32(5))
    return x0 ^ x1


def _gumbel(bits):
    fb = lax.shift_right_logical(bits, jnp.int32(9)) | jnp.int32(0x3F800000)
    u = lax.bitcast_convert_type(fb, jnp.float32) - jnp.float32(1.0)
    inner = -jnp.log(u + jnp.float32(1e-8)) + jnp.float32(1e-8)
    return -jnp.log(inner)


def _tc_body(logits_ref, out_ref, pos_ref, vmax_ref):
    j = pl.program_id(0)

    @pl.when(j == 0)
    def _init():
        vmax_ref[...] = jnp.full((ROWS, 1), -jnp.inf, jnp.float32)
        pos_ref[...] = jnp.zeros((ROWS, 1), jnp.int32)

    c = j * BLOCK_C + lax.broadcasted_iota(jnp.int32, (ROWS, BLOCK_C), 1)
    r = lax.broadcasted_iota(jnp.int32, (ROWS, BLOCK_C), 0)
    flat = r * jnp.int32(COLS) + c
    g = _gumbel(_threefry_bits(flat))
    z = logits_ref[...] + g
    z = jnp.where(c < COLS, z, -jnp.inf)
    m = jnp.max(z, axis=1, keepdims=True)
    a = jnp.min(
        jnp.where(z == m, flat, jnp.int32(0x7FFFFFFF)), axis=1, keepdims=True
    )
    upd = m > vmax_ref[...]
    vmax_ref[...] = jnp.where(upd, m, vmax_ref[...])
    pos_ref[...] = jnp.where(upd, a, pos_ref[...])
    out_ref[...] = jnp.zeros((ROWS, BLOCK_C), jnp.float32)


def _argmax_and_zeros(logits):
    return pl.pallas_call(
        _tc_body,
        grid=(NB,),
        in_specs=[pl.BlockSpec((ROWS, BLOCK_C), lambda j: (0, j))],
        out_specs=[
            pl.BlockSpec((ROWS, BLOCK_C), lambda j: (0, j)),
            pl.BlockSpec((ROWS, 1), lambda j: (0, 0)),
        ],
        out_shape=[
            jax.ShapeDtypeStruct((ROWS, COLS), jnp.float32),
            jax.ShapeDtypeStruct((ROWS, 1), jnp.int32),
        ],
        scratch_shapes=[pltpu.VMEM((ROWS, 1), jnp.float32)],
        compiler_params=pltpu.CompilerParams(
            dimension_semantics=("arbitrary",),
        ),
    )(logits)


@functools.cache
def _make_sc_scatter_ones():
    mesh = plsc.VectorSubcoreMesh(core_axis_name="c", subcore_axis_name="s")

    @functools.partial(
        pl.kernel,
        mesh=mesh,
        scratch_types=[
            pltpu.VMEM((ROWS,), jnp.int32),
            pltpu.VMEM((ROWS,), jnp.float32),
            pltpu.SemaphoreType.DMA,
        ],
    )
    def _sc_scatter_ones(pos_hbm, buf_hbm, pos_v, ones_v, sem):
        cid = lax.axis_index("c")
        sid = lax.axis_index("s")

        @pl.when((cid == 0) & (sid == 0))
        def _():
            pltpu.sync_copy(pos_hbm, pos_v)
            for i in range(ROWS // 16):
                ones_v[pl.ds(16 * i, 16)] = jnp.full((16,), 1.0, jnp.float32)
            pltpu.async_copy(ones_v, buf_hbm.at[pos_v], sem).wait()

    return _sc_scatter_ones


@jax.jit
def kernel(logits):
    zeros2d, pos = _argmax_and_zeros(logits)
    buf = jax.new_ref(zeros2d.reshape(ROWS * COLS))
    _make_sc_scatter_ones()(pos.reshape(ROWS), buf)
    return buf[...].reshape(ROWS, COLS)


# speedup vs baseline: 1.3660x; 1.3660x over previous
import jax
import jax.numpy as jnp
from jax import lax
from jax.experimental import pallas as pl
from jax.experimental.pallas import tpu as pltpu

ROWS = 128
COLS = 100000
BLOCK_C = 2048
NB = (COLS + BLOCK_C - 1) // BLOCK_C

_KS0 = 0
_KS1 = 42
_KS2 = 42 ^ 0x1BD11BDA

_ROT_A = (13, 15, 26, 6)
_ROT_B = (17, 29, 16, 24)


def _rotl(x, d):
    return lax.shift_left(x, jnp.int32(d)) | lax.shift_right_logical(
        x, jnp.int32(32 - d)
    )


def _rounds(x0, x1, rots):
    for d in rots:
        x0 = x0 + x1
        x1 = x0 ^ _rotl(x1, d)
    return x0, x1


def _threefry_bits(flat_idx):
    ks0 = jnp.int32(_KS0)
    ks1 = jnp.int32(_KS1)
    ks2 = jnp.int32(_KS2)
    x0 = jnp.zeros_like(flat_idx) + ks0
    x1 = flat_idx + ks1
    x0, x1 = _rounds(x0, x1, _ROT_A)
    x0 = x0 + ks1
    x1 = x1 + (ks2 + jnp.int32(1))
    x0, x1 = _rounds(x0, x1, _ROT_B)
    x0 = x0 + ks2
    x1 = x1 + (ks0 + jnp.int32(2))
    x0, x1 = _rounds(x0, x1, _ROT_A)
    x0 = x0 + ks0
    x1 = x1 + (ks1 + jnp.int32(3))
    x0, x1 = _rounds(x0, x1, _ROT_B)
    x0 = x0 + ks1
    x1 = x1 + (ks2 + jnp.int32(4))
    x0, x1 = _rounds(x0, x1, _ROT_A)
    x0 = x0 + ks2
    x1 = x1 + (ks0 + jnp.int32(5))
    return x0 ^ x1


def _gumbel(bits):
    fb = lax.shift_right_logical(bits, jnp.int32(9)) | jnp.int32(0x3F800000)
    u = lax.bitcast_convert_type(fb, jnp.float32) - jnp.float32(1.0)
    inner = -jnp.log(u + jnp.float32(1e-8)) + jnp.float32(1e-8)
    return -jnp.log(inner)


def _body(logits_ref, out_ref, pos_ref, vmax_ref):
    j = pl.program_id(0)

    @pl.when(j == 0)
    def _init():
        vmax_ref[...] = jnp.full((ROWS, 1), -jnp.inf, jnp.float32)
        pos_ref[...] = jnp.zeros((ROWS, 1), jnp.int32)

    @pl.when(j < NB)
    def _pass1():
        c = j * BLOCK_C + lax.broadcasted_iota(jnp.int32, (ROWS, BLOCK_C), 1)
        r = lax.broadcasted_iota(jnp.int32, (ROWS, BLOCK_C), 0)
        flat = r * jnp.int32(COLS) + c
        g = _gumbel(_threefry_bits(flat))
        z = logits_ref[...] + g
        z = jnp.where(c < COLS, z, -jnp.inf)
        m = jnp.max(z, axis=1, keepdims=True)
        a = jnp.min(
            jnp.where(z == m, flat, jnp.int32(0x7FFFFFFF)),
            axis=1,
            keepdims=True,
        )
        upd = m > vmax_ref[...]
        vmax_ref[...] = jnp.where(upd, m, vmax_ref[...])
        pos_ref[...] = jnp.where(upd, a, pos_ref[...])

    @pl.when(j >= NB)
    def _pass2():
        jb = j - NB
        c = jb * BLOCK_C + lax.broadcasted_iota(jnp.int32, (ROWS, BLOCK_C), 1)
        r = lax.broadcasted_iota(jnp.int32, (ROWS, BLOCK_C), 0)
        flat = r * jnp.int32(COLS) + c
        out_ref[...] = jnp.where(flat == pos_ref[...], 1.0, 0.0).astype(
            jnp.float32
        )


@jax.jit
def kernel(logits):
    return pl.pallas_call(
        _body,
        grid=(2 * NB,),
        in_specs=[
            pl.BlockSpec(
                (ROWS, BLOCK_C), lambda j: (0, jnp.minimum(j, NB - 1))
            )
        ],
        out_specs=pl.BlockSpec(
            (ROWS, BLOCK_C), lambda j: (0, jnp.maximum(j - NB, 0))
        ),
        out_shape=jax.ShapeDtypeStruct((ROWS, COLS), jnp.float32),
        scratch_shapes=[
            pltpu.VMEM((ROWS, 1), jnp.int32),
            pltpu.VMEM((ROWS, 1), jnp.float32),
        ],
        compiler_params=pltpu.CompilerParams(
            dimension_semantics=("arbitrary",),
        ),
    )(logits)
